# baseline (speedup 1.0000x reference)
import functools

import jax
import jax.numpy as jnp
from jax import lax
from jax.experimental import pallas as pl
from jax.experimental.pallas import tpu as pltpu
from jax.experimental.pallas import tpu_sc as plsc

_B = 100000
_D = 128
_CH = 80
_NCHUNKS = _B // _CH

_info = plsc.get_sparse_core_info()
_NC = _info.num_cores
_NS = _info.num_subcores
_NW = _NC * _NS


def _emb_body(idx_hbm, table_hbm, out_hbm, idx_v, rows_v, sem):
    wid = lax.axis_index("s") * _NC + lax.axis_index("c")
    n = (_NCHUNKS - wid + _NW - 1) // _NW

    def body(t, carry):
        c = wid + t * _NW
        base = pl.multiple_of(c * _CH, 8)
        pltpu.sync_copy(idx_hbm.at[pl.ds(base, _CH)], idx_v)
        pltpu.async_copy(table_hbm.at[idx_v], rows_v, sem).wait()
        pltpu.sync_copy(rows_v, out_hbm.at[pl.ds(base, _CH)])
        return carry

    lax.fori_loop(0, n, body, 0)


def kernel(node_types, type_embeddings):
    node_types = node_types.astype(jnp.int32)
    type_embeddings = type_embeddings.astype(jnp.float32)

    mesh = plsc.VectorSubcoreMesh(core_axis_name="c", subcore_axis_name="s")
    run = functools.partial(
        pl.kernel,
        mesh=mesh,
        out_type=jax.ShapeDtypeStruct((_B, _D), jnp.float32),
        scratch_types=[
            pltpu.VMEM((_CH,), jnp.int32),
            pltpu.VMEM((_CH, _D), jnp.float32),
            pltpu.SemaphoreType.DMA,
        ],
    )(_emb_body)
    return run(node_types, type_embeddings)


# speedup vs baseline: 1.0041x; 1.0041x over previous
import functools

import jax
import jax.numpy as jnp
from jax import lax
from jax.experimental import pallas as pl
from jax.experimental.pallas import tpu as pltpu
from jax.experimental.pallas import tpu_sc as plsc

_B = 100000
_D = 128
_CH = 80
_NCHUNKS = _B // _CH
_BASE_NC = _NCHUNKS // 32

_info = plsc.get_sparse_core_info()
_NC = _info.num_cores
_NS = _info.num_subcores
_NW = _NC * _NS
_EXTRA = _NCHUNKS - _BASE_NC * _NW


def _emb_body(idx_hbm, table_hbm, out_hbm, idx_v, rows_v, sem):
    wid = lax.axis_index("s") * _NC + lax.axis_index("c")
    has_extra = wid < _EXTRA
    n = _BASE_NC + has_extra.astype(jnp.int32)
    start = _BASE_NC * wid + jnp.minimum(wid, _EXTRA)
    base_row = pl.multiple_of(start * _CH, 8)

    pltpu.sync_copy(
        idx_hbm.at[pl.ds(base_row, _BASE_NC * _CH)],
        idx_v.at[pl.ds(0, _BASE_NC * _CH)],
    )

    @pl.when(has_extra)
    def _():
        pltpu.sync_copy(
            idx_hbm.at[pl.ds(base_row + _BASE_NC * _CH, _CH)],
            idx_v.at[pl.ds(_BASE_NC * _CH, _CH)],
        )

    def body(t, carry):
        off = pl.multiple_of(t * _CH, 8)
        pltpu.async_copy(
            table_hbm.at[idx_v.at[pl.ds(off, _CH)]], rows_v, sem
        ).wait()
        pltpu.sync_copy(rows_v, out_hbm.at[pl.ds(base_row + off, _CH)])
        return carry

    lax.fori_loop(0, n, body, 0)


def kernel(node_types, type_embeddings):
    node_types = node_types.astype(jnp.int32)
    type_embeddings = type_embeddings.astype(jnp.float32)

    mesh = plsc.VectorSubcoreMesh(core_axis_name="c", subcore_axis_name="s")
    run = functools.partial(
        pl.kernel,
        mesh=mesh,
        out_type=jax.ShapeDtypeStruct((_B, _D), jnp.float32),
        scratch_types=[
            pltpu.VMEM(((_BASE_NC + 1) * _CH,), jnp.int32),
            pltpu.VMEM((_CH, _D), jnp.float32),
            pltpu.SemaphoreType.DMA,
        ],
    )(_emb_body)
    return run(node_types, type_embeddings)


# speedup vs baseline: 1.0094x; 1.0053x over previous
import functools

import jax
import jax.numpy as jnp
from jax import lax
from jax.experimental import pallas as pl
from jax.experimental.pallas import tpu as pltpu
from jax.experimental.pallas import tpu_sc as plsc

_B = 100000
_D = 128
_CH = 80
_NCHUNKS = _B // _CH
_BASE_NC = _NCHUNKS // 32

_info = plsc.get_sparse_core_info()
_NC = _info.num_cores
_NS = _info.num_subcores
_NW = _NC * _NS
_EXTRA = _NCHUNKS - _BASE_NC * _NW


def _emb_body(idx_hbm, table_hbm, out_hbm, idx_v, rows0, rows1,
              gsem0, gsem1, wsem0, wsem1):
    wid = lax.axis_index("s") * _NC + lax.axis_index("c")
    has_extra = wid < _EXTRA
    n = _BASE_NC + has_extra.astype(jnp.int32)
    start = _BASE_NC * wid + jnp.minimum(wid, _EXTRA)
    base_row = pl.multiple_of(start * _CH, 8)

    pltpu.sync_copy(
        idx_hbm.at[pl.ds(base_row, _BASE_NC * _CH)],
        idx_v.at[pl.ds(0, _BASE_NC * _CH)],
    )

    @pl.when(has_extra)
    def _():
        pltpu.sync_copy(
            idx_hbm.at[pl.ds(base_row + _BASE_NC * _CH, _CH)],
            idx_v.at[pl.ds(_BASE_NC * _CH, _CH)],
        )

    def fire_gather(t, buf, gsem):
        off = pl.multiple_of(t * _CH, 8)
        pltpu.async_copy(table_hbm.at[idx_v.at[pl.ds(off, _CH)]], buf, gsem)

    def wait_gather(buf, gsem):
        pltpu.make_async_copy(
            table_hbm.at[idx_v.at[pl.ds(0, _CH)]], buf, gsem
        ).wait()

    def fire_write(t, buf, wsem):
        off = pl.multiple_of(t * _CH, 8)
        pltpu.async_copy(buf, out_hbm.at[pl.ds(base_row + off, _CH)], wsem)

    def wait_write(buf, wsem):
        pltpu.make_async_copy(
            buf, out_hbm.at[pl.ds(base_row, _CH)], wsem
        ).wait()

    fire_gather(0, rows0, gsem0)

    def body(t, carry):
        is0 = (t % 2) == 0

        @pl.when((t >= 1) & is0)
        def _():
            wait_write(rows1, wsem1)

        @pl.when((t >= 1) & jnp.logical_not(is0))
        def _():
            wait_write(rows0, wsem0)

        @pl.when((t + 1 < n) & is0)
        def _():
            fire_gather(t + 1, rows1, gsem1)

        @pl.when((t + 1 < n) & jnp.logical_not(is0))
        def _():
            fire_gather(t + 1, rows0, gsem0)

        @pl.when(is0)
        def _():
            wait_gather(rows0, gsem0)
            fire_write(t, rows0, wsem0)

        @pl.when(jnp.logical_not(is0))
        def _():
            wait_gather(rows1, gsem1)
            fire_write(t, rows1, wsem1)

        return carry

    lax.fori_loop(0, n, body, 0)

    last_is0 = ((n - 1) % 2) == 0

    @pl.when(last_is0)
    def _():
        wait_write(rows0, wsem0)

    @pl.when(jnp.logical_not(last_is0))
    def _():
        wait_write(rows1, wsem1)


def kernel(node_types, type_embeddings):
    node_types = node_types.astype(jnp.int32)
    type_embeddings = type_embeddings.astype(jnp.float32)

    mesh = plsc.VectorSubcoreMesh(core_axis_name="c", subcore_axis_name="s")
    run = functools.partial(
        pl.kernel,
        mesh=mesh,
        out_type=jax.ShapeDtypeStruct((_B, _D), jnp.float32),
        scratch_types=[
            pltpu.VMEM(((_BASE_NC + 1) * _CH,), jnp.int32),
            pltpu.VMEM((_CH, _D), jnp.float32),
            pltpu.VMEM((_CH, _D), jnp.float32),
            pltpu.SemaphoreType.DMA,
            pltpu.SemaphoreType.DMA,
            pltpu.SemaphoreType.DMA,
            pltpu.SemaphoreType.DMA,
        ],
    )(_emb_body)
    return run(node_types, type_embeddings)


# speedup vs baseline: 8.3312x; 8.2535x over previous
import functools

import jax
import jax.numpy as jnp
from jax import lax
from jax.experimental import pallas as pl
from jax.experimental.pallas import tpu as pltpu
from jax.experimental.pallas import tpu_sc as plsc

_B = 100000
_D = 128
_CH = 80
_NCHUNKS = _B // _CH
_BASE_NC = _NCHUNKS // 32

_info = plsc.get_sparse_core_info()
_NC = _info.num_cores
_NS = _info.num_subcores
_NW = _NC * _NS
_EXTRA = _NCHUNKS - _BASE_NC * _NW


def _emb_body(idx_hbm, table_hbm, out_hbm, idx_v, rows0, rows1, table_sh,
              gsem0, gsem1, wsem0, wsem1):
    sid = lax.axis_index("s")
    wid = sid * _NC + lax.axis_index("c")

    @pl.when(sid == 0)
    def _():
        pltpu.sync_copy(table_hbm, table_sh)

    plsc.subcore_barrier()
    has_extra = wid < _EXTRA
    n = _BASE_NC + has_extra.astype(jnp.int32)
    start = _BASE_NC * wid + jnp.minimum(wid, _EXTRA)
    base_row = pl.multiple_of(start * _CH, 8)

    pltpu.sync_copy(
        idx_hbm.at[pl.ds(base_row, _BASE_NC * _CH)],
        idx_v.at[pl.ds(0, _BASE_NC * _CH)],
    )

    @pl.when(has_extra)
    def _():
        pltpu.sync_copy(
            idx_hbm.at[pl.ds(base_row + _BASE_NC * _CH, _CH)],
            idx_v.at[pl.ds(_BASE_NC * _CH, _CH)],
        )

    def fire_gather(t, buf, gsem):
        off = pl.multiple_of(t * _CH, 8)
        pltpu.async_copy(table_sh.at[idx_v.at[pl.ds(off, _CH)]], buf, gsem)

    def wait_gather(buf, gsem):
        pltpu.make_async_copy(
            table_sh.at[idx_v.at[pl.ds(0, _CH)]], buf, gsem
        ).wait()

    def fire_write(t, buf, wsem):
        off = pl.multiple_of(t * _CH, 8)
        pltpu.async_copy(buf, out_hbm.at[pl.ds(base_row + off, _CH)], wsem)

    def wait_write(buf, wsem):
        pltpu.make_async_copy(
            buf, out_hbm.at[pl.ds(base_row, _CH)], wsem
        ).wait()

    fire_gather(0, rows0, gsem0)

    def body(t, carry):
        is0 = (t % 2) == 0

        @pl.when((t >= 1) & is0)
        def _():
            wait_write(rows1, wsem1)

        @pl.when((t >= 1) & jnp.logical_not(is0))
        def _():
            wait_write(rows0, wsem0)

        @pl.when((t + 1 < n) & is0)
        def _():
            fire_gather(t + 1, rows1, gsem1)

        @pl.when((t + 1 < n) & jnp.logical_not(is0))
        def _():
            fire_gather(t + 1, rows0, gsem0)

        @pl.when(is0)
        def _():
            wait_gather(rows0, gsem0)
            fire_write(t, rows0, wsem0)

        @pl.when(jnp.logical_not(is0))
        def _():
            wait_gather(rows1, gsem1)
            fire_write(t, rows1, wsem1)

        return carry

    lax.fori_loop(0, n, body, 0)

    last_is0 = ((n - 1) % 2) == 0

    @pl.when(last_is0)
    def _():
        wait_write(rows0, wsem0)

    @pl.when(jnp.logical_not(last_is0))
    def _():
        wait_write(rows1, wsem1)


def kernel(node_types, type_embeddings):
    node_types = node_types.astype(jnp.int32)
    type_embeddings = type_embeddings.astype(jnp.float32)

    mesh = plsc.VectorSubcoreMesh(core_axis_name="c", subcore_axis_name="s")
    run = functools.partial(
        pl.kernel,
        mesh=mesh,
        out_type=jax.ShapeDtypeStruct((_B, _D), jnp.float32),
        scratch_types=[
            pltpu.VMEM(((_BASE_NC + 1) * _CH,), jnp.int32),
            pltpu.VMEM((_CH, _D), jnp.float32),
            pltpu.VMEM((_CH, _D), jnp.float32),
            pltpu.VMEM_SHARED((16, _D), jnp.float32),
            pltpu.SemaphoreType.DMA,
            pltpu.SemaphoreType.DMA,
            pltpu.SemaphoreType.DMA,
            pltpu.SemaphoreType.DMA,
        ],
    )(_emb_body)
    return run(node_types, type_embeddings)


# speedup vs baseline: 8.4000x; 1.0083x over previous
import functools

import jax
import jax.numpy as jnp
from jax import lax
from jax.experimental import pallas as pl
from jax.experimental.pallas import tpu as pltpu
from jax.experimental.pallas import tpu_sc as plsc

_B = 100000
_D = 128
_SUB = 80
_CH = 160
_NCHUNKS = _B // _CH
_BASE_NC = _NCHUNKS // 32

_info = plsc.get_sparse_core_info()
_NC = _info.num_cores
_NS = _info.num_subcores
_NW = _NC * _NS
_EXTRA = _NCHUNKS - _BASE_NC * _NW


def _emb_body(idx_hbm, table_hbm, out_hbm, idx_v, rows0, rows1, table_sh,
              gsem0, gsem1, wsem0, wsem1):
    sid = lax.axis_index("s")
    wid = sid * _NC + lax.axis_index("c")

    @pl.when(sid == 0)
    def _():
        pltpu.sync_copy(table_hbm, table_sh)

    plsc.subcore_barrier()

    has_extra = wid < _EXTRA
    n = _BASE_NC + has_extra.astype(jnp.int32)
    start = _BASE_NC * wid + jnp.minimum(wid, _EXTRA)
    base_row = pl.multiple_of(start * _CH, 8)

    pltpu.sync_copy(
        idx_hbm.at[pl.ds(base_row, _BASE_NC * _CH)],
        idx_v.at[pl.ds(0, _BASE_NC * _CH)],
    )

    @pl.when(has_extra)
    def _():
        pltpu.sync_copy(
            idx_hbm.at[pl.ds(base_row + _BASE_NC * _CH, _CH)],
            idx_v.at[pl.ds(_BASE_NC * _CH, _CH)],
        )

    def fire_gather(t, buf, gsem):
        off = pl.multiple_of(t * _CH, 8)
        pltpu.async_copy(
            table_sh.at[idx_v.at[pl.ds(off, _SUB)]],
            buf.at[pl.ds(0, _SUB)], gsem)
        pltpu.async_copy(
            table_sh.at[idx_v.at[pl.ds(off + _SUB, _SUB)]],
            buf.at[pl.ds(_SUB, _SUB)], gsem)

    def wait_gather(buf, gsem):
        pltpu.make_async_copy(
            table_sh.at[idx_v.at[pl.ds(0, _SUB)]],
            buf.at[pl.ds(0, _SUB)], gsem
        ).wait()
        pltpu.make_async_copy(
            table_sh.at[idx_v.at[pl.ds(0, _SUB)]],
            buf.at[pl.ds(_SUB, _SUB)], gsem
        ).wait()

    def fire_write(t, buf, wsem):
        off = pl.multiple_of(t * _CH, 8)
        pltpu.async_copy(buf, out_hbm.at[pl.ds(base_row + off, _CH)], wsem)

    def wait_write(buf, wsem):
        pltpu.make_async_copy(
            buf, out_hbm.at[pl.ds(base_row, _CH)], wsem
        ).wait()

    fire_gather(0, rows0, gsem0)

    def body(t, carry):
        is0 = (t % 2) == 0

        @pl.when((t >= 1) & is0)
        def _():
            wait_write(rows1, wsem1)

        @pl.when((t >= 1) & jnp.logical_not(is0))
        def _():
            wait_write(rows0, wsem0)

        @pl.when((t + 1 < n) & is0)
        def _():
            fire_gather(t + 1, rows1, gsem1)

        @pl.when((t + 1 < n) & jnp.logical_not(is0))
        def _():
            fire_gather(t + 1, rows0, gsem0)

        @pl.when(is0)
        def _():
            wait_gather(rows0, gsem0)
            fire_write(t, rows0, wsem0)

        @pl.when(jnp.logical_not(is0))
        def _():
            wait_gather(rows1, gsem1)
            fire_write(t, rows1, wsem1)

        return carry

    lax.fori_loop(0, n, body, 0)

    last_is0 = ((n - 1) % 2) == 0

    @pl.when(last_is0)
    def _():
        wait_write(rows0, wsem0)

    @pl.when(jnp.logical_not(last_is0))
    def _():
        wait_write(rows1, wsem1)


def kernel(node_types, type_embeddings):
    node_types = node_types.astype(jnp.int32)
    type_embeddings = type_embeddings.astype(jnp.float32)

    mesh = plsc.VectorSubcoreMesh(core_axis_name="c", subcore_axis_name="s")
    run = functools.partial(
        pl.kernel,
        mesh=mesh,
        out_type=jax.ShapeDtypeStruct((_B, _D), jnp.float32),
        scratch_types=[
            pltpu.VMEM(((_BASE_NC + 1) * _CH,), jnp.int32),
            pltpu.VMEM((_CH, _D), jnp.float32),
            pltpu.VMEM((_CH, _D), jnp.float32),
            pltpu.VMEM_SHARED((16, _D), jnp.float32),
            pltpu.SemaphoreType.DMA,
            pltpu.SemaphoreType.DMA,
            pltpu.SemaphoreType.DMA,
            pltpu.SemaphoreType.DMA,
        ],
    )(_emb_body)
    return run(node_types, type_embeddings)


# speedup vs baseline: 8.4271x; 1.0032x over previous
import functools

import jax
import jax.numpy as jnp
from jax import lax
from jax.experimental import pallas as pl
from jax.experimental.pallas import tpu as pltpu
from jax.experimental.pallas import tpu_sc as plsc

_B = 100000
_D = 128
_SUB = 80
_CH = 160
_NCHUNKS = _B // _CH
_BASE_NC = _NCHUNKS // 32

_info = plsc.get_sparse_core_info()
_NC = _info.num_cores
_NS = _info.num_subcores
_NW = _NC * _NS
_EXTRA = _NCHUNKS - _BASE_NC * _NW


def _emb_body(idx_hbm, table_hbm, out_hbm, idx_v, rows0, rows1, table_sh,
              gsem0, gsem1, wsem0, wsem1):
    sid = lax.axis_index("s")
    wid = sid * _NC + lax.axis_index("c")

    @pl.when(sid == 0)
    def _():
        pltpu.sync_copy(table_hbm, table_sh)

    plsc.subcore_barrier()

    has_extra = wid < _EXTRA
    n = _BASE_NC + has_extra.astype(jnp.int32)
    start = _BASE_NC * wid + jnp.minimum(wid, _EXTRA)
    base_row = pl.multiple_of(start * _CH, 8)

    pltpu.sync_copy(
        idx_hbm.at[pl.ds(base_row, _BASE_NC * _CH)],
        idx_v.at[pl.ds(0, _BASE_NC * _CH)],
    )

    @pl.when(has_extra)
    def _():
        pltpu.sync_copy(
            idx_hbm.at[pl.ds(base_row + _BASE_NC * _CH, _CH)],
            idx_v.at[pl.ds(_BASE_NC * _CH, _CH)],
        )

    def fire_gather(t, buf, gsem):
        off = pl.multiple_of(t * _CH, 8)
        pltpu.async_copy(
            table_sh.at[idx_v.at[pl.ds(off, _CH)]], buf, gsem)

    def wait_gather(buf, gsem):
        pltpu.make_async_copy(
            table_sh.at[idx_v.at[pl.ds(0, _CH)]], buf, gsem
        ).wait()

    def fire_write(t, buf, wsem):
        off = pl.multiple_of(t * _CH, 8)
        pltpu.async_copy(buf, out_hbm.at[pl.ds(base_row + off, _CH)], wsem)

    def wait_write(buf, wsem):
        pltpu.make_async_copy(
            buf, out_hbm.at[pl.ds(base_row, _CH)], wsem
        ).wait()

    fire_gather(0, rows0, gsem0)

    def body(t, carry):
        is0 = (t % 2) == 0

        @pl.when((t >= 1) & is0)
        def _():
            wait_write(rows1, wsem1)

        @pl.when((t >= 1) & jnp.logical_not(is0))
        def _():
            wait_write(rows0, wsem0)

        @pl.when((t + 1 < n) & is0)
        def _():
            fire_gather(t + 1, rows1, gsem1)

        @pl.when((t + 1 < n) & jnp.logical_not(is0))
        def _():
            fire_gather(t + 1, rows0, gsem0)

        @pl.when(is0)
        def _():
            wait_gather(rows0, gsem0)
            fire_write(t, rows0, wsem0)

        @pl.when(jnp.logical_not(is0))
        def _():
            wait_gather(rows1, gsem1)
            fire_write(t, rows1, wsem1)

        return carry

    lax.fori_loop(0, n, body, 0)

    last_is0 = ((n - 1) % 2) == 0

    @pl.when(last_is0)
    def _():
        wait_write(rows0, wsem0)

    @pl.when(jnp.logical_not(last_is0))
    def _():
        wait_write(rows1, wsem1)


def kernel(node_types, type_embeddings):
    node_types = node_types.astype(jnp.int32)
    type_embeddings = type_embeddings.astype(jnp.float32)

    mesh = plsc.VectorSubcoreMesh(core_axis_name="c", subcore_axis_name="s")
    run = functools.partial(
        pl.kernel,
        mesh=mesh,
        out_type=jax.ShapeDtypeStruct((_B, _D), jnp.float32),
        scratch_types=[
            pltpu.VMEM(((_BASE_NC + 1) * _CH,), jnp.int32),
            pltpu.VMEM((_CH, _D), jnp.float32),
            pltpu.VMEM((_CH, _D), jnp.float32),
            pltpu.VMEM_SHARED((16, _D), jnp.float32),
            pltpu.SemaphoreType.DMA,
            pltpu.SemaphoreType.DMA,
            pltpu.SemaphoreType.DMA,
            pltpu.SemaphoreType.DMA,
        ],
    )(_emb_body)
    return run(node_types, type_embeddings)
